# baseline (speedup 1.0000x reference)
import functools

import jax
import jax.numpy as jnp
from jax import lax
from jax.experimental import pallas as pl
from jax.experimental.pallas import tpu as pltpu
from jax.experimental.pallas import tpu_sc as plsc

VOCAB = 100000
EMBED_DIM = 128
BATCH = 4096
SEQ_LEN = 200

B = BATCH * SEQ_LEN
NC, NS = 2, 16
NW = NC * NS
B_PER_W = B // NW
CHUNK = 128
N_CHUNK = B_PER_W // CHUNK
SUP = 2
BIG = SUP * CHUNK
N_SUP = N_CHUNK // SUP

_mesh = plsc.VectorSubcoreMesh(core_axis_name="c", subcore_axis_name="s")


@functools.partial(
    pl.kernel,
    mesh=_mesh,
    out_type=jax.ShapeDtypeStruct((B, EMBED_DIM), jnp.float32),
    scratch_types=[
        pltpu.VMEM((N_CHUNK, CHUNK), jnp.int32),
        pltpu.VMEM((2, BIG, EMBED_DIM), jnp.float32),
        pltpu.SemaphoreType.DMA,
        pltpu.SemaphoreType.DMA,
        pltpu.SemaphoreType.DMA,
        pltpu.SemaphoreType.DMA,
    ],
)
def _gather_kernel(idx_hbm, table_hbm, out_hbm, idx_v, rows_v, g0, g1, w0, w1):
    wid = lax.axis_index("s") * NC + lax.axis_index("c")
    row0 = wid * N_CHUNK
    base = wid * B_PER_W
    gsem = [g0, g1]
    wsem = [w0, w1]

    pltpu.sync_copy(idx_hbm.at[pl.ds(row0, N_CHUNK)], idx_v)

    def gstart(s, buf):
        for q in range(SUP):
            pltpu.async_copy(
                table_hbm.at[idx_v.at[s * SUP + q]],
                rows_v.at[buf, pl.ds(q * CHUNK, CHUNK)],
                gsem[buf],
            )

    def gwait(s, buf):
        for q in range(SUP):
            pltpu.make_async_copy(
                table_hbm.at[idx_v.at[s * SUP + q]],
                rows_v.at[buf, pl.ds(q * CHUNK, CHUNK)],
                gsem[buf],
            ).wait()

    def wstart(s, buf):
        pltpu.async_copy(
            rows_v.at[buf], out_hbm.at[pl.ds(base + s * BIG, BIG)], wsem[buf]
        )

    def wwait(s, buf):
        pltpu.make_async_copy(
            rows_v.at[buf], out_hbm.at[pl.ds(base + s * BIG, BIG)], wsem[buf]
        ).wait()

    gstart(0, 0)

    def body(g, carry):
        for b in range(2):
            s = 2 * g + b

            @pl.when(s > 0)
            def _(s=s, b=b):
                wwait(s - 1, 1 - b)

            @pl.when(s + 1 < N_SUP)
            def _(s=s, b=b):
                gstart(s + 1, 1 - b)

            gwait(s, b)
            wstart(s, b)

        return carry

    lax.fori_loop(0, N_SUP // 2, body, 0)

    wwait(N_SUP - 1, (N_SUP - 1) % 2)


def kernel(np_batch, table):
    idx = np_batch.astype(jnp.int32).reshape(B // CHUNK, CHUNK)
    out = _gather_kernel(idx, table)
    return out.reshape(BATCH, SEQ_LEN, EMBED_DIM)


# speedup vs baseline: 1.0010x; 1.0010x over previous
import functools

import jax
import jax.numpy as jnp
from jax import lax
from jax.experimental import pallas as pl
from jax.experimental.pallas import tpu as pltpu
from jax.experimental.pallas import tpu_sc as plsc

VOCAB = 100000
EMBED_DIM = 128
BATCH = 4096
SEQ_LEN = 200

B = BATCH * SEQ_LEN
NC, NS = 2, 16
NW = NC * NS
B_PER_W = B // NW
CHUNK = 128
N_CHUNK = B_PER_W // CHUNK
SUP = 2
BIG = SUP * CHUNK
N_SUP = N_CHUNK // SUP

_mesh = plsc.VectorSubcoreMesh(core_axis_name="c", subcore_axis_name="s")


@functools.partial(
    pl.kernel,
    mesh=_mesh,
    out_type=jax.ShapeDtypeStruct((B, EMBED_DIM), jnp.float32),
    scratch_types=[
        pltpu.VMEM((N_CHUNK, CHUNK), jnp.int32),
        pltpu.VMEM((2, BIG, EMBED_DIM), jnp.float32),
        pltpu.SemaphoreType.DMA,
        pltpu.SemaphoreType.DMA,
        pltpu.SemaphoreType.DMA,
        pltpu.SemaphoreType.DMA,
    ],
)
def _gather_kernel(idx_hbm, table_hbm, out_hbm, idx_v, rows_v, g0, g1, w0, w1):
    wid = lax.axis_index("c") * NS + lax.axis_index("s")
    row0 = wid * N_CHUNK
    base = wid * B_PER_W
    gsem = [g0, g1]
    wsem = [w0, w1]

    pltpu.sync_copy(idx_hbm.at[pl.ds(row0, N_CHUNK)], idx_v)

    def gstart(s, buf):
        for q in range(SUP):
            pltpu.async_copy(
                table_hbm.at[idx_v.at[s * SUP + q]],
                rows_v.at[buf, pl.ds(q * CHUNK, CHUNK)],
                gsem[buf],
            )

    def gwait(s, buf):
        for q in range(SUP):
            pltpu.make_async_copy(
                table_hbm.at[idx_v.at[s * SUP + q]],
                rows_v.at[buf, pl.ds(q * CHUNK, CHUNK)],
                gsem[buf],
            ).wait()

    def wstart(s, buf):
        pltpu.async_copy(
            rows_v.at[buf], out_hbm.at[pl.ds(base + s * BIG, BIG)], wsem[buf]
        )

    def wwait(s, buf):
        pltpu.make_async_copy(
            rows_v.at[buf], out_hbm.at[pl.ds(base + s * BIG, BIG)], wsem[buf]
        ).wait()

    gstart(0, 0)

    def body(g, carry):
        for b in range(2):
            s = 2 * g + b

            @pl.when(s > 0)
            def _(s=s, b=b):
                wwait(s - 1, 1 - b)

            @pl.when(s + 1 < N_SUP)
            def _(s=s, b=b):
                gstart(s + 1, 1 - b)

            gwait(s, b)
            wstart(s, b)

        return carry

    lax.fori_loop(0, N_SUP // 2, body, 0)

    wwait(N_SUP - 1, (N_SUP - 1) % 2)


def kernel(np_batch, table):
    idx = np_batch.astype(jnp.int32).reshape(B // CHUNK, CHUNK)
    out = _gather_kernel(idx, table)
    return out.reshape(BATCH, SEQ_LEN, EMBED_DIM)


# speedup vs baseline: 1.0165x; 1.0155x over previous
import functools

import jax
import jax.numpy as jnp
from jax import lax
from jax.experimental import pallas as pl
from jax.experimental.pallas import tpu as pltpu
from jax.experimental.pallas import tpu_sc as plsc

VOCAB = 100000
EMBED_DIM = 128
BATCH = 4096
SEQ_LEN = 200

B = BATCH * SEQ_LEN
NC, NS = 2, 16
NW = NC * NS
CHUNK = 128
N_CHUNK_TOT = B // CHUNK
SUP = 2
BIG = SUP * CHUNK

EXTRA = 8
NCH0 = N_CHUNK_TOT // NW + EXTRA
NCH1 = N_CHUNK_TOT // NW - EXTRA
STAGE = NCH0

_mesh = plsc.VectorSubcoreMesh(core_axis_name="c", subcore_axis_name="s")


@functools.partial(
    pl.kernel,
    mesh=_mesh,
    out_type=jax.ShapeDtypeStruct((B, EMBED_DIM), jnp.float32),
    scratch_types=[
        pltpu.VMEM((STAGE, CHUNK), jnp.int32),
        pltpu.VMEM((2, BIG, EMBED_DIM), jnp.float32),
        pltpu.SemaphoreType.DMA,
        pltpu.SemaphoreType.DMA,
        pltpu.SemaphoreType.DMA,
        pltpu.SemaphoreType.DMA,
    ],
)
def _gather_kernel(idx_hbm, table_hbm, out_hbm, idx_v, rows_v, g0, g1, w0, w1):
    cid = lax.axis_index("c")
    sid = lax.axis_index("s")
    row0 = jnp.where(cid == 0, sid * NCH0, NS * NCH0 + sid * NCH1)
    n_chunk = jnp.where(cid == 0, NCH0, NCH1)
    n_sup = n_chunk // SUP
    base = row0 * CHUNK
    start0 = jnp.minimum(row0, N_CHUNK_TOT - STAGE)
    off = row0 - start0
    gsem = [g0, g1]
    wsem = [w0, w1]

    pltpu.sync_copy(idx_hbm.at[pl.ds(start0, STAGE)], idx_v)

    def gstart(s, buf):
        for q in range(SUP):
            pltpu.async_copy(
                table_hbm.at[idx_v.at[off + s * SUP + q]],
                rows_v.at[buf, pl.ds(q * CHUNK, CHUNK)],
                gsem[buf],
            )

    def gwait(s, buf):
        for q in range(SUP):
            pltpu.make_async_copy(
                table_hbm.at[idx_v.at[off + s * SUP + q]],
                rows_v.at[buf, pl.ds(q * CHUNK, CHUNK)],
                gsem[buf],
            ).wait()

    def wstart(s, buf):
        pltpu.async_copy(
            rows_v.at[buf], out_hbm.at[pl.ds(base + s * BIG, BIG)], wsem[buf]
        )

    def wwait(s, buf):
        pltpu.make_async_copy(
            rows_v.at[buf], out_hbm.at[pl.ds(base + s * BIG, BIG)], wsem[buf]
        ).wait()

    gstart(0, 0)

    def body(g, carry):
        for b in range(2):
            s = 2 * g + b

            @pl.when(s > 0)
            def _(s=s, b=b):
                wwait(s - 1, 1 - b)

            @pl.when(s + 1 < n_sup)
            def _(s=s, b=b):
                gstart(s + 1, 1 - b)

            gwait(s, b)
            wstart(s, b)

        return carry

    lax.fori_loop(0, n_sup // 2, body, 0)

    wwait(n_sup - 1, 1)


def kernel(np_batch, table):
    idx = np_batch.astype(jnp.int32).reshape(B // CHUNK, CHUNK)
    out = _gather_kernel(idx, table)
    return out.reshape(BATCH, SEQ_LEN, EMBED_DIM)
